# baseline (speedup 1.0000x reference)
import functools

import jax
import jax.numpy as jnp
from jax import lax
from jax.experimental import pallas as pl
from jax.experimental.pallas import tpu as pltpu
from jax.experimental.pallas import tpu_sc as plsc

N = 1024
M = 1024
D = 128
B = 4096

UL = 3072
UH = B - UL

NC = 2
NS = 16
NW = NC * NS
BPW = UL // NW

RB_A = 256
RB_B = 512
GRID_A = UH // RB_A
GRID_B = UL // RB_B


@functools.cache
def _sc_gather_ulo():
    mesh = plsc.VectorSubcoreMesh(
        core_axis_name="c", subcore_axis_name="s",
        num_cores=NC, num_subcores=NS)

    @functools.partial(
        pl.kernel,
        mesh=mesh,
        out_type=jax.ShapeDtypeStruct((UL, D), jnp.float32),
        scratch_types=[
            pltpu.VMEM((BPW,), jnp.int32),
            pltpu.VMEM((BPW, D), jnp.float32),
            pltpu.SemaphoreType.DMA,
            pltpu.SemaphoreType.DMA,
        ],
    )
    def gather(user_hbm, uid_hbm, u_out, uidx_v, urows_v, sem_a, sem_b):
        wid = lax.axis_index("s") * NC + lax.axis_index("c")
        base = wid * BPW
        pltpu.async_copy(uid_hbm.at[pl.ds(base, BPW)], uidx_v, sem_a).wait()
        pltpu.async_copy(user_hbm.at[uidx_v], urows_v, sem_b).wait()
        pltpu.async_copy(urows_v, u_out.at[pl.ds(base, BPW)], sem_a).wait()

    return gather


def _bottom_body(ut_ref, it_ref, uid_ref, iid_ref, big_ref, vb_out_ref,
                 oh_ref, vb_ref, uhb_ref):
    i = pl.program_id(0)

    @pl.when(i == 0)
    def _():
        iota_i = lax.broadcasted_iota(jnp.int32, (M, B), 0)
        oh_ref[...] = jnp.where(
            iota_i == iid_ref[...], 1.0, 0.0).astype(jnp.bfloat16)
        vb_ref[...] = lax.dot_general(
            oh_ref[...], it_ref[...].astype(jnp.bfloat16),
            (((0,), (0,)), ((), ())),
            preferred_element_type=jnp.float32).astype(jnp.bfloat16)
        iota_u = lax.broadcasted_iota(jnp.int32, (N, UH), 0)
        oh_u = jnp.where(
            iota_u == uid_ref[pl.ds(UL, UH)], 1.0, 0.0).astype(jnp.bfloat16)
        uhb_ref[...] = lax.dot_general(
            oh_u, ut_ref[...].astype(jnp.bfloat16), (((0,), (0,)), ((), ())),
            preferred_element_type=jnp.float32).astype(jnp.bfloat16)

    big_ref[...] = lax.dot_general(
        uhb_ref[pl.ds(i * RB_A, RB_A), :], vb_ref[...], (((1,), (1,)), ((), ())),
        preferred_element_type=jnp.float32)
    vb_out_ref[...] = vb_ref[pl.ds(i * (B // GRID_A), B // GRID_A), :]


@functools.cache
def _tc_bottom():
    return pl.pallas_call(
        _bottom_body,
        grid=(GRID_A,),
        in_specs=[pl.BlockSpec((N, D), lambda i: (0, 0)),
                  pl.BlockSpec((M, D), lambda i: (0, 0)),
                  pl.BlockSpec((B,), lambda i: (0,)),
                  pl.BlockSpec((B,), lambda i: (0,))],
        out_specs=[pl.BlockSpec((RB_A, B), lambda i: (UL // RB_A + i, 0)),
                   pl.BlockSpec((B // GRID_A, D), lambda i: (i, 0))],
        out_shape=[jax.ShapeDtypeStruct((B, B), jnp.float32),
                   jax.ShapeDtypeStruct((B, D), jnp.bfloat16)],
        scratch_shapes=[pltpu.VMEM((M, B), jnp.bfloat16),
                        pltpu.VMEM((B, D), jnp.bfloat16),
                        pltpu.VMEM((UH, D), jnp.bfloat16)],
    )


def _top_body(ulo_ref, vb_ref, big_ref, o_ref):
    o_ref[...] = lax.dot_general(
        ulo_ref[...].astype(jnp.bfloat16), vb_ref[...],
        (((1,), (1,)), ((), ())),
        preferred_element_type=jnp.float32)


@functools.cache
def _tc_top():
    return pl.pallas_call(
        _top_body,
        grid=(GRID_B,),
        in_specs=[pl.BlockSpec((RB_B, D), lambda i: (i, 0)),
                  pl.BlockSpec((B, D), lambda i: (0, 0)),
                  pl.BlockSpec(memory_space=pl.ANY)],
        out_specs=pl.BlockSpec((RB_B, B), lambda i: (i, 0)),
        out_shape=jax.ShapeDtypeStruct((B, B), jnp.float32),
        input_output_aliases={2: 0},
    )


def kernel(user_table, item_table, user_ids, item_ids):
    big, vb = _tc_bottom()(user_table, item_table, user_ids, item_ids)
    u_lo = _sc_gather_ulo()(user_table, user_ids)
    return _tc_top()(u_lo, vb, big)


# speedup vs baseline: 1.0179x; 1.0179x over previous
import functools

import jax
import jax.numpy as jnp
from jax import lax
from jax.experimental import pallas as pl
from jax.experimental.pallas import tpu as pltpu
from jax.experimental.pallas import tpu_sc as plsc

N = 1024
M = 1024
D = 128
B = 4096

UL = 3072
UH = B - UL

NC = 1
NS = 16
NW = NC * NS
BPW = UL // NW

RB_A = 256
RB_B = 512
GRID_A = UH // RB_A
GRID_B = UL // RB_B


@functools.cache
def _sc_gather_ulo():
    mesh = plsc.VectorSubcoreMesh(
        core_axis_name="c", subcore_axis_name="s",
        num_cores=NC, num_subcores=NS)

    @functools.partial(
        pl.kernel,
        mesh=mesh,
        out_type=jax.ShapeDtypeStruct((UL, D), jnp.float32),
        scratch_types=[
            pltpu.VMEM((BPW,), jnp.int32),
            pltpu.VMEM((BPW, D), jnp.float32),
            pltpu.SemaphoreType.DMA,
            pltpu.SemaphoreType.DMA,
        ],
    )
    def gather(user_hbm, uid_hbm, u_out, uidx_v, urows_v, sem_a, sem_b):
        wid = lax.axis_index("s") * NC + lax.axis_index("c")
        base = wid * BPW
        pltpu.async_copy(uid_hbm.at[pl.ds(base, BPW)], uidx_v, sem_a).wait()
        pltpu.async_copy(user_hbm.at[uidx_v], urows_v, sem_b).wait()
        pltpu.async_copy(urows_v, u_out.at[pl.ds(base, BPW)], sem_a).wait()

    return gather


def _bottom_body(ut_ref, it_ref, uid_ref, iid_ref, big_ref, vb_out_ref,
                 oh_ref, vb_ref, uhb_ref):
    i = pl.program_id(0)

    @pl.when(i == 0)
    def _():
        iota_i = lax.broadcasted_iota(jnp.int32, (M, B), 0)
        oh_ref[...] = jnp.where(
            iota_i == iid_ref[...], 1.0, 0.0).astype(jnp.bfloat16)
        vb_ref[...] = lax.dot_general(
            oh_ref[...], it_ref[...].astype(jnp.bfloat16),
            (((0,), (0,)), ((), ())),
            preferred_element_type=jnp.float32).astype(jnp.bfloat16)
        iota_u = lax.broadcasted_iota(jnp.int32, (N, UH), 0)
        oh_u = jnp.where(
            iota_u == uid_ref[pl.ds(UL, UH)], 1.0, 0.0).astype(jnp.bfloat16)
        uhb_ref[...] = lax.dot_general(
            oh_u, ut_ref[...].astype(jnp.bfloat16), (((0,), (0,)), ((), ())),
            preferred_element_type=jnp.float32).astype(jnp.bfloat16)

    big_ref[...] = lax.dot_general(
        uhb_ref[pl.ds(i * RB_A, RB_A), :], vb_ref[...], (((1,), (1,)), ((), ())),
        preferred_element_type=jnp.float32)
    vb_out_ref[...] = vb_ref[pl.ds(i * (B // GRID_A), B // GRID_A), :]


@functools.cache
def _tc_bottom():
    return pl.pallas_call(
        _bottom_body,
        grid=(GRID_A,),
        in_specs=[pl.BlockSpec((N, D), lambda i: (0, 0)),
                  pl.BlockSpec((M, D), lambda i: (0, 0)),
                  pl.BlockSpec((B,), lambda i: (0,)),
                  pl.BlockSpec((B,), lambda i: (0,))],
        out_specs=[pl.BlockSpec((RB_A, B), lambda i: (UL // RB_A + i, 0)),
                   pl.BlockSpec((B // GRID_A, D), lambda i: (i, 0))],
        out_shape=[jax.ShapeDtypeStruct((B, B), jnp.float32),
                   jax.ShapeDtypeStruct((B, D), jnp.bfloat16)],
        scratch_shapes=[pltpu.VMEM((M, B), jnp.bfloat16),
                        pltpu.VMEM((B, D), jnp.bfloat16),
                        pltpu.VMEM((UH, D), jnp.bfloat16)],
    )


def _top_body(ulo_ref, vb_ref, big_ref, o_ref):
    o_ref[...] = lax.dot_general(
        ulo_ref[...].astype(jnp.bfloat16), vb_ref[...],
        (((1,), (1,)), ((), ())),
        preferred_element_type=jnp.float32)


@functools.cache
def _tc_top():
    return pl.pallas_call(
        _top_body,
        grid=(GRID_B,),
        in_specs=[pl.BlockSpec((RB_B, D), lambda i: (i, 0)),
                  pl.BlockSpec((B, D), lambda i: (0, 0)),
                  pl.BlockSpec(memory_space=pl.ANY)],
        out_specs=pl.BlockSpec((RB_B, B), lambda i: (i, 0)),
        out_shape=jax.ShapeDtypeStruct((B, B), jnp.float32),
        input_output_aliases={2: 0},
    )


def kernel(user_table, item_table, user_ids, item_ids):
    big, vb = _tc_bottom()(user_table, item_table, user_ids, item_ids)
    u_lo = _sc_gather_ulo()(user_table, user_ids)
    return _tc_top()(u_lo, vb, big)


# speedup vs baseline: 1.0220x; 1.0040x over previous
import functools

import jax
import jax.numpy as jnp
from jax import lax
from jax.experimental import pallas as pl
from jax.experimental.pallas import tpu as pltpu
from jax.experimental.pallas import tpu_sc as plsc

N = 1024
M = 1024
D = 128
B = 4096

UL = 3584
UH = B - UL

NC = 1
NS = 16
NW = NC * NS
BPW = UL // NW

RB_A = 256
RB_B = 512
GRID_A = UH // RB_A
GRID_B = UL // RB_B


@functools.cache
def _sc_gather_ulo():
    mesh = plsc.VectorSubcoreMesh(
        core_axis_name="c", subcore_axis_name="s",
        num_cores=NC, num_subcores=NS)

    @functools.partial(
        pl.kernel,
        mesh=mesh,
        out_type=jax.ShapeDtypeStruct((UL, D), jnp.float32),
        scratch_types=[
            pltpu.VMEM((BPW,), jnp.int32),
            pltpu.VMEM((BPW, D), jnp.float32),
            pltpu.SemaphoreType.DMA,
            pltpu.SemaphoreType.DMA,
        ],
    )
    def gather(user_hbm, uid_hbm, u_out, uidx_v, urows_v, sem_a, sem_b):
        wid = lax.axis_index("s") * NC + lax.axis_index("c")
        base = wid * BPW
        pltpu.async_copy(uid_hbm.at[pl.ds(base, BPW)], uidx_v, sem_a).wait()
        pltpu.async_copy(user_hbm.at[uidx_v], urows_v, sem_b).wait()
        pltpu.async_copy(urows_v, u_out.at[pl.ds(base, BPW)], sem_a).wait()

    return gather


def _bottom_body(ut_ref, it_ref, uid_ref, iid_ref, big_ref, vb_out_ref,
                 oh_ref, vb_ref, uhb_ref):
    i = pl.program_id(0)

    @pl.when(i == 0)
    def _():
        iota_i = lax.broadcasted_iota(jnp.int32, (M, B), 0)
        oh_ref[...] = jnp.where(
            iota_i == iid_ref[...], 1.0, 0.0).astype(jnp.bfloat16)
        vb_ref[...] = lax.dot_general(
            oh_ref[...], it_ref[...].astype(jnp.bfloat16),
            (((0,), (0,)), ((), ())),
            preferred_element_type=jnp.float32).astype(jnp.bfloat16)
        iota_u = lax.broadcasted_iota(jnp.int32, (N, UH), 0)
        oh_u = jnp.where(
            iota_u == uid_ref[pl.ds(UL, UH)], 1.0, 0.0).astype(jnp.bfloat16)
        uhb_ref[...] = lax.dot_general(
            oh_u, ut_ref[...].astype(jnp.bfloat16), (((0,), (0,)), ((), ())),
            preferred_element_type=jnp.float32).astype(jnp.bfloat16)

    big_ref[...] = lax.dot_general(
        uhb_ref[pl.ds(i * RB_A, RB_A), :], vb_ref[...], (((1,), (1,)), ((), ())),
        preferred_element_type=jnp.float32)
    vb_out_ref[...] = vb_ref[pl.ds(i * (B // GRID_A), B // GRID_A), :]


@functools.cache
def _tc_bottom():
    return pl.pallas_call(
        _bottom_body,
        grid=(GRID_A,),
        in_specs=[pl.BlockSpec((N, D), lambda i: (0, 0)),
                  pl.BlockSpec((M, D), lambda i: (0, 0)),
                  pl.BlockSpec((B,), lambda i: (0,)),
                  pl.BlockSpec((B,), lambda i: (0,))],
        out_specs=[pl.BlockSpec((RB_A, B), lambda i: (UL // RB_A + i, 0)),
                   pl.BlockSpec((B // GRID_A, D), lambda i: (i, 0))],
        out_shape=[jax.ShapeDtypeStruct((B, B), jnp.float32),
                   jax.ShapeDtypeStruct((B, D), jnp.bfloat16)],
        scratch_shapes=[pltpu.VMEM((M, B), jnp.bfloat16),
                        pltpu.VMEM((B, D), jnp.bfloat16),
                        pltpu.VMEM((UH, D), jnp.bfloat16)],
    )


def _top_body(ulo_ref, vb_ref, big_ref, o_ref):
    o_ref[...] = lax.dot_general(
        ulo_ref[...].astype(jnp.bfloat16), vb_ref[...],
        (((1,), (1,)), ((), ())),
        preferred_element_type=jnp.float32)


@functools.cache
def _tc_top():
    return pl.pallas_call(
        _top_body,
        grid=(GRID_B,),
        in_specs=[pl.BlockSpec((RB_B, D), lambda i: (i, 0)),
                  pl.BlockSpec((B, D), lambda i: (0, 0)),
                  pl.BlockSpec(memory_space=pl.ANY)],
        out_specs=pl.BlockSpec((RB_B, B), lambda i: (i, 0)),
        out_shape=jax.ShapeDtypeStruct((B, B), jnp.float32),
        input_output_aliases={2: 0},
    )


def kernel(user_table, item_table, user_ids, item_ids):
    big, vb = _tc_bottom()(user_table, item_table, user_ids, item_ids)
    u_lo = _sc_gather_ulo()(user_table, user_ids)
    return _tc_top()(u_lo, vb, big)
